# baseline (speedup 1.0000x reference)
import functools

import jax
import jax.numpy as jnp
from jax import lax
from jax.experimental import pallas as pl
from jax.experimental.pallas import tpu as pltpu
from jax.experimental.pallas import tpu_sc as plsc

K_TOTAL = 8192
D_DIM = 256
K_CHUNK = 2048


G_ROWS = 32


def _argmin_body(z_ref, emb_ref, out_ref):
    zb = z_ref[0]
    T = zb.shape[1]
    z2 = jnp.sum(zb * zb, axis=0, keepdims=True)
    zb2 = zb + zb
    n_folds = K_CHUNK // G_ROWS

    def step(c, carry):
        rminv, rmini = carry
        ech = emb_ref[pl.ds(c * K_CHUNK, K_CHUNK), :]
        e2 = jnp.sum(ech * ech, axis=1, keepdims=True)
        m2 = jnp.dot(ech, zb2, preferred_element_type=jnp.float32)
        for f in range(n_folds):
            dsub = (z2 + e2[f * G_ROWS:(f + 1) * G_ROWS]) \
                - m2[f * G_ROWS:(f + 1) * G_ROWS]
            take = dsub < rminv
            fidx = jnp.float32(c * n_folds + f)
            rminv = jnp.where(take, dsub, rminv)
            rmini = jnp.where(take, fidx, rmini)
        return (rminv, rmini)

    rminv0 = jnp.full((G_ROWS, T), jnp.inf, dtype=jnp.float32)
    rmini0 = jnp.zeros((G_ROWS, T), dtype=jnp.float32)
    rminv, rmini = lax.fori_loop(0, K_TOTAL // K_CHUNK, step, (rminv0, rmini0))

    gmin = jnp.min(rminv, axis=0, keepdims=True)
    g_iota = lax.broadcasted_iota(jnp.int32, (G_ROWS, T), 0).astype(jnp.float32)
    k_cand = rmini * jnp.float32(G_ROWS) + g_iota
    cidx = jnp.min(
        jnp.where(rminv == gmin, k_cand, jnp.float32(2**30)),
        axis=0, keepdims=True,
    )
    out_ref[0] = cidx.astype(jnp.int32)


def _tc_argmin(z, embeddings):
    B, D, T = z.shape
    return pl.pallas_call(
        _argmin_body,
        grid=(B,),
        in_specs=[
            pl.BlockSpec((1, D, T), lambda b: (b, 0, 0)),
            pl.BlockSpec((K_TOTAL, D), lambda b: (0, 0)),
        ],
        out_specs=pl.BlockSpec((1, 1, T), lambda b: (b, 0, 0)),
        out_shape=jax.ShapeDtypeStruct((B, 1, T), jnp.int32),
        compiler_params=pltpu.CompilerParams(
            dimension_semantics=("parallel",),
        ),
    )(z, embeddings)


def _sc_gather(table, idx):
    N = idx.shape[0]
    D = table.shape[1]
    NC, NS = 2, 16
    NW = NC * NS
    b_per_w = N // NW
    mesh = plsc.VectorSubcoreMesh(core_axis_name="c", subcore_axis_name="s")

    @functools.partial(
        pl.kernel,
        mesh=mesh,
        out_type=jax.ShapeDtypeStruct((N, D), jnp.float32),
        scratch_types=[
            pltpu.VMEM((b_per_w,), jnp.int32),
            pltpu.VMEM((b_per_w, D), jnp.float32),
            pltpu.SemaphoreType.DMA,
        ],
    )
    def gather_k(table_hbm, idx_hbm, out_hbm, idx_v, rows_v, sem):
        wid = lax.axis_index("s") * NC + lax.axis_index("c")
        base = wid * b_per_w
        pltpu.sync_copy(idx_hbm.at[pl.ds(base, b_per_w)], idx_v)
        pltpu.async_copy(table_hbm.at[idx_v], rows_v, sem).wait()
        pltpu.sync_copy(rows_v, out_hbm.at[pl.ds(base, b_per_w)])

    return gather_k(table, idx)


def kernel(z, embeddings):
    B, D, T = z.shape
    idx = _tc_argmin(z, embeddings).reshape(B * T)
    rows = _sc_gather(embeddings, idx)
    vq = jnp.transpose(rows.reshape(B, T, D), (0, 2, 1))
    return z + (vq - z)


# speedup vs baseline: 1.0019x; 1.0019x over previous
import functools

import jax
import jax.numpy as jnp
from jax import lax
from jax.experimental import pallas as pl
from jax.experimental.pallas import tpu as pltpu
from jax.experimental.pallas import tpu_sc as plsc

K_TOTAL = 8192
D_DIM = 256
K_CHUNK = 2048


G_ROWS = 8


def _argmin_body(z_ref, emb_ref, out_ref):
    zb = z_ref[0]
    T = zb.shape[1]
    z2 = jnp.sum(zb * zb, axis=0, keepdims=True)
    zb2 = zb + zb
    n_folds = K_CHUNK // G_ROWS

    def step(c, carry):
        rminv, rmini = carry
        ech = emb_ref[pl.ds(c * K_CHUNK, K_CHUNK), :]
        e2 = jnp.sum(ech * ech, axis=1, keepdims=True)
        m2 = jnp.dot(ech, zb2, preferred_element_type=jnp.float32)
        for f in range(n_folds):
            dsub = (z2 + e2[f * G_ROWS:(f + 1) * G_ROWS]) \
                - m2[f * G_ROWS:(f + 1) * G_ROWS]
            take = dsub < rminv
            fidx = jnp.float32(c * n_folds + f)
            rminv = jnp.where(take, dsub, rminv)
            rmini = jnp.where(take, fidx, rmini)
        return (rminv, rmini)

    rminv0 = jnp.full((G_ROWS, T), jnp.inf, dtype=jnp.float32)
    rmini0 = jnp.zeros((G_ROWS, T), dtype=jnp.float32)
    rminv, rmini = lax.fori_loop(0, K_TOTAL // K_CHUNK, step, (rminv0, rmini0))

    gmin = jnp.min(rminv, axis=0, keepdims=True)
    g_iota = lax.broadcasted_iota(jnp.int32, (G_ROWS, T), 0).astype(jnp.float32)
    k_cand = rmini * jnp.float32(G_ROWS) + g_iota
    cidx = jnp.min(
        jnp.where(rminv == gmin, k_cand, jnp.float32(2**30)),
        axis=0, keepdims=True,
    )
    out_ref[0] = cidx.astype(jnp.int32)


def _tc_argmin(z, embeddings):
    B, D, T = z.shape
    return pl.pallas_call(
        _argmin_body,
        grid=(B,),
        in_specs=[
            pl.BlockSpec((1, D, T), lambda b: (b, 0, 0)),
            pl.BlockSpec((K_TOTAL, D), lambda b: (0, 0)),
        ],
        out_specs=pl.BlockSpec((1, 1, T), lambda b: (b, 0, 0)),
        out_shape=jax.ShapeDtypeStruct((B, 1, T), jnp.int32),
        compiler_params=pltpu.CompilerParams(
            dimension_semantics=("parallel",),
        ),
    )(z, embeddings)


def _sc_gather(table, idx):
    N = idx.shape[0]
    D = table.shape[1]
    NC, NS = 2, 16
    NW = NC * NS
    b_per_w = N // NW
    mesh = plsc.VectorSubcoreMesh(core_axis_name="c", subcore_axis_name="s")

    @functools.partial(
        pl.kernel,
        mesh=mesh,
        out_type=jax.ShapeDtypeStruct((N, D), jnp.float32),
        scratch_types=[
            pltpu.VMEM((b_per_w,), jnp.int32),
            pltpu.VMEM((b_per_w, D), jnp.float32),
            pltpu.SemaphoreType.DMA,
        ],
    )
    def gather_k(table_hbm, idx_hbm, out_hbm, idx_v, rows_v, sem):
        wid = lax.axis_index("s") * NC + lax.axis_index("c")
        base = wid * b_per_w
        pltpu.sync_copy(idx_hbm.at[pl.ds(base, b_per_w)], idx_v)
        pltpu.async_copy(table_hbm.at[idx_v], rows_v, sem).wait()
        pltpu.sync_copy(rows_v, out_hbm.at[pl.ds(base, b_per_w)])

    return gather_k(table, idx)


def kernel(z, embeddings):
    B, D, T = z.shape
    idx = _tc_argmin(z, embeddings).reshape(B * T)
    rows = _sc_gather(embeddings, idx)
    vq = jnp.transpose(rows.reshape(B, T, D), (0, 2, 1))
    return z + (vq - z)


# speedup vs baseline: 1.0113x; 1.0094x over previous
import functools

import jax
import jax.numpy as jnp
from jax import lax
from jax.experimental import pallas as pl
from jax.experimental.pallas import tpu as pltpu
from jax.experimental.pallas import tpu_sc as plsc

K_TOTAL = 8192
D_DIM = 256
K_CHUNK = 2048


G_ROWS = 16


def _argmin_body(z_ref, emb_ref, out_ref, e2_ref):
    zb = z_ref[0]
    T = zb.shape[1]
    z2 = jnp.sum(zb * zb, axis=0, keepdims=True)
    zb2 = zb + zb
    n_folds = K_CHUNK // G_ROWS

    @pl.when(pl.program_id(0) == 0)
    def _():
        def init(c, _):
            ech = emb_ref[pl.ds(c * K_CHUNK, K_CHUNK), :]
            e2_ref[pl.ds(c * K_CHUNK, K_CHUNK), :] = jnp.sum(
                ech * ech, axis=1, keepdims=True
            )
            return 0
        lax.fori_loop(0, K_TOTAL // K_CHUNK, init, 0)

    def step(c, carry):
        rminv, rmini = carry
        ech = emb_ref[pl.ds(c * K_CHUNK, K_CHUNK), :]
        e2 = e2_ref[pl.ds(c * K_CHUNK, K_CHUNK), :]
        m2 = jnp.dot(ech, zb2, preferred_element_type=jnp.float32)
        for f in range(n_folds):
            dsub = (z2 + e2[f * G_ROWS:(f + 1) * G_ROWS]) \
                - m2[f * G_ROWS:(f + 1) * G_ROWS]
            take = dsub < rminv
            fidx = jnp.float32(c * n_folds + f)
            rminv = jnp.where(take, dsub, rminv)
            rmini = jnp.where(take, fidx, rmini)
        return (rminv, rmini)

    rminv0 = jnp.full((G_ROWS, T), jnp.inf, dtype=jnp.float32)
    rmini0 = jnp.zeros((G_ROWS, T), dtype=jnp.float32)
    rminv, rmini = lax.fori_loop(0, K_TOTAL // K_CHUNK, step, (rminv0, rmini0))

    gmin = jnp.min(rminv, axis=0, keepdims=True)
    g_iota = lax.broadcasted_iota(jnp.int32, (G_ROWS, T), 0).astype(jnp.float32)
    k_cand = rmini * jnp.float32(G_ROWS) + g_iota
    cidx = jnp.min(
        jnp.where(rminv == gmin, k_cand, jnp.float32(2**30)),
        axis=0, keepdims=True,
    )
    out_ref[0] = cidx.astype(jnp.int32)


def _tc_argmin(z, embeddings):
    B, D, T = z.shape
    return pl.pallas_call(
        _argmin_body,
        grid=(B,),
        in_specs=[
            pl.BlockSpec((1, D, T), lambda b: (b, 0, 0)),
            pl.BlockSpec((K_TOTAL, D), lambda b: (0, 0)),
        ],
        out_specs=pl.BlockSpec((1, 1, T), lambda b: (b, 0, 0)),
        out_shape=jax.ShapeDtypeStruct((B, 1, T), jnp.int32),
        scratch_shapes=[pltpu.VMEM((K_TOTAL, 1), jnp.float32)],
        compiler_params=pltpu.CompilerParams(
            dimension_semantics=("arbitrary",),
        ),
    )(z, embeddings)


def _sc_gather(table, idx):
    N = idx.shape[0]
    D = table.shape[1]
    NC, NS = 2, 16
    NW = NC * NS
    b_per_w = N // NW
    mesh = plsc.VectorSubcoreMesh(core_axis_name="c", subcore_axis_name="s")

    @functools.partial(
        pl.kernel,
        mesh=mesh,
        out_type=jax.ShapeDtypeStruct((N, D), jnp.float32),
        scratch_types=[
            pltpu.VMEM((b_per_w,), jnp.int32),
            pltpu.VMEM((b_per_w, D), jnp.float32),
            pltpu.SemaphoreType.DMA,
        ],
    )
    def gather_k(table_hbm, idx_hbm, out_hbm, idx_v, rows_v, sem):
        wid = lax.axis_index("s") * NC + lax.axis_index("c")
        base = wid * b_per_w
        pltpu.sync_copy(idx_hbm.at[pl.ds(base, b_per_w)], idx_v)
        pltpu.async_copy(table_hbm.at[idx_v], rows_v, sem).wait()
        pltpu.sync_copy(rows_v, out_hbm.at[pl.ds(base, b_per_w)])

    return gather_k(table, idx)


def kernel(z, embeddings):
    B, D, T = z.shape
    idx = _tc_argmin(z, embeddings).reshape(B * T)
    rows = _sc_gather(embeddings, idx)
    vq = jnp.transpose(rows.reshape(B, T, D), (0, 2, 1))
    return z + (vq - z)


# speedup vs baseline: 1.0209x; 1.0095x over previous
import functools

import jax
import jax.numpy as jnp
from jax import lax
from jax.experimental import pallas as pl
from jax.experimental.pallas import tpu as pltpu
from jax.experimental.pallas import tpu_sc as plsc

K_TOTAL = 8192
D_DIM = 256
K_CHUNK = 2048


G_ROWS = 16


def _argmin_body(z_ref, emb_ref, out_ref):
    zb = z_ref[0]
    T = zb.shape[1]
    z2 = jnp.sum(zb * zb, axis=0, keepdims=True)
    zb2 = zb + zb
    n_folds = K_CHUNK // G_ROWS

    def step(c, carry):
        rminv, rmini = carry
        ech = emb_ref[pl.ds(c * K_CHUNK, K_CHUNK), :]
        e2 = jnp.sum(ech * ech, axis=1, keepdims=True)
        m2 = jnp.dot(ech, zb2, preferred_element_type=jnp.float32)
        for f in range(n_folds):
            dsub = (z2 + e2[f * G_ROWS:(f + 1) * G_ROWS]) \
                - m2[f * G_ROWS:(f + 1) * G_ROWS]
            take = dsub < rminv
            fidx = jnp.float32(c * n_folds + f)
            rminv = jnp.where(take, dsub, rminv)
            rmini = jnp.where(take, fidx, rmini)
        return (rminv, rmini)

    rminv0 = jnp.full((G_ROWS, T), jnp.inf, dtype=jnp.float32)
    rmini0 = jnp.zeros((G_ROWS, T), dtype=jnp.float32)
    rminv, rmini = lax.fori_loop(0, K_TOTAL // K_CHUNK, step, (rminv0, rmini0))

    gmin = jnp.min(rminv, axis=0, keepdims=True)
    g_iota = lax.broadcasted_iota(jnp.int32, (G_ROWS, T), 0).astype(jnp.float32)
    k_cand = rmini * jnp.float32(G_ROWS) + g_iota
    cidx = jnp.min(
        jnp.where(rminv == gmin, k_cand, jnp.float32(2**30)),
        axis=0, keepdims=True,
    )
    out_ref[0] = cidx.astype(jnp.int32)


def _tc_argmin(z, embeddings):
    B, D, T = z.shape
    return pl.pallas_call(
        _argmin_body,
        grid=(B,),
        in_specs=[
            pl.BlockSpec((1, D, T), lambda b: (b, 0, 0)),
            pl.BlockSpec((K_TOTAL, D), lambda b: (0, 0)),
        ],
        out_specs=pl.BlockSpec((1, 1, T), lambda b: (b, 0, 0)),
        out_shape=jax.ShapeDtypeStruct((B, 1, T), jnp.int32),
        compiler_params=pltpu.CompilerParams(
            dimension_semantics=("parallel",),
        ),
    )(z, embeddings)


def _sc_gather(table, idx):
    N = idx.shape[0]
    D = table.shape[1]
    NC, NS = 2, 16
    NW = NC * NS
    b_per_w = N // NW
    mesh = plsc.VectorSubcoreMesh(core_axis_name="c", subcore_axis_name="s")

    @functools.partial(
        pl.kernel,
        mesh=mesh,
        out_type=jax.ShapeDtypeStruct((N, D), jnp.float32),
        scratch_types=[
            pltpu.VMEM((b_per_w,), jnp.int32),
            pltpu.VMEM((b_per_w, D), jnp.float32),
            pltpu.SemaphoreType.DMA,
        ],
    )
    def gather_k(table_hbm, idx_hbm, out_hbm, idx_v, rows_v, sem):
        wid = lax.axis_index("s") * NC + lax.axis_index("c")
        base = wid * b_per_w
        pltpu.sync_copy(idx_hbm.at[pl.ds(base, b_per_w)], idx_v)
        pltpu.async_copy(table_hbm.at[idx_v], rows_v, sem).wait()
        pltpu.sync_copy(rows_v, out_hbm.at[pl.ds(base, b_per_w)])

    return gather_k(table, idx)


def kernel(z, embeddings):
    B, D, T = z.shape
    idx = _tc_argmin(z, embeddings).reshape(B * T)
    rows = _sc_gather(embeddings, idx)
    vq = jnp.transpose(rows.reshape(B, T, D), (0, 2, 1))
    return z + (vq - z)
